# baseline (speedup 1.0000x reference)
import jax
import jax.numpy as jnp
from jax import lax
from jax.experimental import pallas as pl
from jax.experimental.pallas import tpu as pltpu
from jax.experimental.pallas import tpu_sc as plsc
import functools

NC = 2
NS = 16
NW = NC * NS
CHUNK = 128


@functools.lru_cache(maxsize=None)
def _build_sc_call(B, D, n_neg):
    assert B % (NW * CHUNK) == 0
    bpw = B // NW
    nch = bpw // CHUNK
    ngrp = bpw // 16

    mesh = plsc.VectorSubcoreMesh(core_axis_name="c", subcore_axis_name="s")

    f32 = jnp.float32
    out_type = (
        jax.ShapeDtypeStruct((B,), f32),
        jax.ShapeDtypeStruct((B,), f32),
        jax.ShapeDtypeStruct((B,), f32),
        jax.ShapeDtypeStruct((B,), f32),
        jax.ShapeDtypeStruct((n_neg, B), f32),
    )
    scratch_types = [
        pltpu.VMEM((nch, CHUNK), jnp.int32),
        pltpu.VMEM((nch, CHUNK), jnp.int32),
        pltpu.VMEM((n_neg, nch, CHUNK), jnp.int32),
        pltpu.VMEM((bpw, D), f32),
        pltpu.VMEM((bpw, D), f32),
        pltpu.VMEM((n_neg, bpw, D), f32),
        pltpu.VMEM((bpw,), f32),
        pltpu.VMEM((bpw,), f32),
        pltpu.VMEM((bpw,), f32),
        pltpu.VMEM((bpw,), f32),
        pltpu.VMEM((n_neg, bpw), f32),
        pltpu.SemaphoreType.DMA,
    ]

    @functools.partial(
        pl.kernel, out_type=out_type, mesh=mesh,
        scratch_types=scratch_types,
        compiler_params=pltpu.CompilerParams(
            needs_layout_passes=False, use_tc_tiling_on_sc=False))
    def sc_kernel(users_hbm, pos_hbm, negT_hbm, uf_hbm, if_hbm,
                  pos_out, qu_out, ppi_out, npi_out, neg_out,
                  uidx, pidx, nidx, urows, prows, nrows,
                  pos_v, qu_v, ppi_v, npi_v, neg_v, sem):
        wid = lax.axis_index("s") * NC + lax.axis_index("c")
        base = wid * bpw
        crow = wid * nch

        stage = [
            pltpu.async_copy(users_hbm.at[pl.ds(crow, nch)], uidx, sem),
            pltpu.async_copy(pos_hbm.at[pl.ds(crow, nch)], pidx, sem),
        ]
        for j in range(n_neg):
            stage.append(pltpu.async_copy(
                negT_hbm.at[j, pl.ds(crow, nch)], nidx.at[j], sem))
        for dsc in stage:
            dsc.wait()

        gathers = []
        for c in range(nch):
            dst = pl.ds(c * CHUNK, CHUNK)
            gathers.append(pltpu.async_copy(
                uf_hbm.at[uidx.at[c]], urows.at[dst], sem))
            gathers.append(pltpu.async_copy(
                if_hbm.at[pidx.at[c]], prows.at[dst], sem))
            for j in range(n_neg):
                gathers.append(pltpu.async_copy(
                    if_hbm.at[nidx.at[j, c]], nrows.at[j, dst], sem))
        for dsc in gathers:
            dsc.wait()

        lane = lax.iota(jnp.int32, 16)
        zero = jnp.zeros((16,), f32)

        def group(g, carry):
            rows = g * 16 + lane
            acc_pp = zero
            acc_qu = zero
            acc_ppi = zero
            acc_npi = zero
            acc_n = [zero] * n_neg
            for d in range(D):
                dcol = jnp.full((16,), d, jnp.int32)
                u = plsc.load_gather(urows, [rows, dcol])
                ip = plsc.load_gather(prows, [rows, dcol])
                acc_pp = acc_pp + u * ip
                acc_qu = acc_qu + u * u
                acc_ppi = acc_ppi + ip * ip
                for j in range(n_neg):
                    nj = plsc.load_gather(nrows.at[j], [rows, dcol])
                    acc_n[j] = acc_n[j] + u * nj
                    if j == n_neg - 1:
                        acc_npi = acc_npi + nj * nj
            sl = pl.ds(g * 16, 16)
            pos_v[sl] = acc_pp
            qu_v[sl] = acc_qu
            ppi_v[sl] = acc_ppi
            npi_v[sl] = acc_npi
            for j in range(n_neg):
                neg_v[j, sl] = acc_n[j]
            return carry

        lax.fori_loop(0, ngrp, group, 0)

        out_sl = pl.ds(base, bpw)
        pltpu.sync_copy(pos_v, pos_out.at[out_sl])
        pltpu.sync_copy(qu_v, qu_out.at[out_sl])
        pltpu.sync_copy(ppi_v, ppi_out.at[out_sl])
        pltpu.sync_copy(npi_v, npi_out.at[out_sl])
        for j in range(n_neg):
            pltpu.sync_copy(neg_v.at[j], neg_out.at[j, out_sl])

    return sc_kernel


def kernel(users, pos_items, neg_items, user_factors, item_factors):
    B = users.shape[0]
    n_neg = neg_items.shape[1]
    D = user_factors.shape[1]

    users_v = users.astype(jnp.int32).reshape(B // CHUNK, CHUNK)
    pos_v = pos_items.astype(jnp.int32).reshape(B // CHUNK, CHUNK)
    negT = neg_items.astype(jnp.int32).T.reshape(n_neg, B // CHUNK, CHUNK)

    sc_call = _build_sc_call(B, D, n_neg)
    pos_preds, qu, ppi, npi, neg_preds = sc_call(
        users_v, pos_v, negT, user_factors, item_factors)

    acc_p = jnp.tile(pos_preds, n_neg)
    acc_n = neg_preds.reshape(-1)
    return ((acc_p, acc_n), (qu, ppi, qu, npi))


# speedup vs baseline: 1.0037x; 1.0037x over previous
import jax
import jax.numpy as jnp
from jax import lax
from jax.experimental import pallas as pl
from jax.experimental.pallas import tpu as pltpu
from jax.experimental.pallas import tpu_sc as plsc
import functools

NC = 2
NS = 16
NW = NC * NS
CHUNK = 128


@functools.lru_cache(maxsize=None)
def _build_sc_call(B, D, n_neg):
    assert B % (NW * CHUNK) == 0
    bpw = B // NW
    nch = bpw // CHUNK
    nnch = nch * n_neg
    ngrp = bpw // 16

    mesh = plsc.VectorSubcoreMesh(core_axis_name="c", subcore_axis_name="s")

    f32 = jnp.float32
    out_type = (
        jax.ShapeDtypeStruct((n_neg, B), f32),
        jax.ShapeDtypeStruct((n_neg, B), f32),
        jax.ShapeDtypeStruct((B,), f32),
        jax.ShapeDtypeStruct((B,), f32),
        jax.ShapeDtypeStruct((B,), f32),
    )
    scratch_types = [
        pltpu.VMEM((nch, CHUNK), jnp.int32),
        pltpu.VMEM((nch, CHUNK), jnp.int32),
        pltpu.VMEM((nnch, CHUNK), jnp.int32),
        pltpu.VMEM((bpw, D), f32),
        pltpu.VMEM((bpw, D), f32),
        pltpu.VMEM((bpw * n_neg, D), f32),
        pltpu.VMEM((bpw,), f32),
        pltpu.VMEM((bpw,), f32),
        pltpu.VMEM((bpw,), f32),
        pltpu.VMEM((bpw,), f32),
        pltpu.VMEM((n_neg, bpw), f32),
        pltpu.SemaphoreType.DMA,
    ]

    @functools.partial(
        pl.kernel, out_type=out_type, mesh=mesh,
        scratch_types=scratch_types,
        compiler_params=pltpu.CompilerParams(
            needs_layout_passes=False, use_tc_tiling_on_sc=False))
    def sc_kernel(users_hbm, pos_hbm, neg_hbm, uf_hbm, if_hbm,
                  accp_out, accn_out, qu_out, ppi_out, npi_out,
                  uidx, pidx, nidx, urows, prows, nrows,
                  pos_v, qu_v, ppi_v, npi_v, neg_v, sem):
        wid = lax.axis_index("s") * NC + lax.axis_index("c")
        base = wid * bpw
        crow = wid * nch

        stage = [
            pltpu.async_copy(users_hbm.at[pl.ds(crow, nch)], uidx, sem),
            pltpu.async_copy(pos_hbm.at[pl.ds(crow, nch)], pidx, sem),
            pltpu.async_copy(neg_hbm.at[pl.ds(wid * nnch, nnch)], nidx, sem),
        ]
        for dsc in stage:
            dsc.wait()

        gathers = []
        for c in range(nch):
            dst = pl.ds(c * CHUNK, CHUNK)
            gathers.append(pltpu.async_copy(
                uf_hbm.at[uidx.at[c]], urows.at[dst], sem))
            gathers.append(pltpu.async_copy(
                if_hbm.at[pidx.at[c]], prows.at[dst], sem))
        for c in range(nnch):
            gathers.append(pltpu.async_copy(
                if_hbm.at[nidx.at[c]], nrows.at[pl.ds(c * CHUNK, CHUNK)], sem))
        for dsc in gathers:
            dsc.wait()

        lane = lax.iota(jnp.int32, 16)
        zero = jnp.zeros((16,), f32)

        def group(g, carry):
            rows = g * 16 + lane
            nrow_base = rows * n_neg
            acc_pp = zero
            acc_qu = zero
            acc_ppi = zero
            acc_npi = zero
            acc_n = [zero] * n_neg
            for d in range(D):
                dcol = jnp.full((16,), d, jnp.int32)
                u = plsc.load_gather(urows, [rows, dcol])
                ip = plsc.load_gather(prows, [rows, dcol])
                acc_pp = acc_pp + u * ip
                acc_qu = acc_qu + u * u
                acc_ppi = acc_ppi + ip * ip
                for j in range(n_neg):
                    nj = plsc.load_gather(nrows, [nrow_base + j, dcol])
                    acc_n[j] = acc_n[j] + u * nj
                    if j == n_neg - 1:
                        acc_npi = acc_npi + nj * nj
            sl = pl.ds(g * 16, 16)
            pos_v[sl] = acc_pp
            qu_v[sl] = acc_qu
            ppi_v[sl] = acc_ppi
            npi_v[sl] = acc_npi
            for j in range(n_neg):
                neg_v[j, sl] = acc_n[j]
            return carry

        lax.fori_loop(0, ngrp, group, 0)

        out_sl = pl.ds(base, bpw)
        pltpu.sync_copy(qu_v, qu_out.at[out_sl])
        pltpu.sync_copy(ppi_v, ppi_out.at[out_sl])
        pltpu.sync_copy(npi_v, npi_out.at[out_sl])
        for j in range(n_neg):
            pltpu.sync_copy(pos_v, accp_out.at[j, out_sl])
            pltpu.sync_copy(neg_v.at[j], accn_out.at[j, out_sl])

    return sc_kernel


def kernel(users, pos_items, neg_items, user_factors, item_factors):
    B = users.shape[0]
    n_neg = neg_items.shape[1]
    D = user_factors.shape[1]

    users_v = users.astype(jnp.int32).reshape(B // CHUNK, CHUNK)
    pos_v = pos_items.astype(jnp.int32).reshape(B // CHUNK, CHUNK)
    neg_v = neg_items.astype(jnp.int32).reshape(B * n_neg // CHUNK, CHUNK)

    sc_call = _build_sc_call(B, D, n_neg)
    accp, accn, qu, ppi, npi = sc_call(
        users_v, pos_v, neg_v, user_factors, item_factors)

    return ((accp.reshape(-1), accn.reshape(-1)), (qu, ppi, qu, npi))
